# baseline (speedup 1.0000x reference)
import jax
import jax.numpy as jnp
from jax import lax
from jax.experimental import pallas as pl
from jax.experimental.pallas import tpu as pltpu
from jax.experimental.pallas import tpu_sc as plsc

_N_NODES = 50000
_N_EDGES = 1600000
_N_GRAPHS = 16
_N_STARTS = 64
_LANES = 16
_NUM_TILES = 32
_CHUNK = 6400
_N_CHUNKS = _N_EDGES // _CHUNK
_ROUNDS = -(-_N_CHUNKS // _NUM_TILES)
_START_BIT = 1 << 30
_GID_MASK = _START_BIT - 1


def _body(scores, ei, batch, gid, starts, ct, pt, stc, out,
          table_v, starts_v, ct_v, pt_v, st_v,
          ei_v0, b_v0, sc_v0, ei_v1, b_v1, sc_v1, o_v0, o_v1,
          sem0, sem1, osem0, osem1):
    wid = lax.axis_index("s") * 2 + lax.axis_index("c")

    pltpu.sync_copy(gid, table_v)
    pltpu.sync_copy(starts, starts_v)
    pltpu.sync_copy(ct, ct_v)
    pltpu.sync_copy(pt, pt_v)
    pltpu.sync_copy(stc, st_v)

    for j in range(_N_STARTS // _LANES):
        sidx = starts_v[pl.ds(j * _LANES, _LANES)]
        cur = plsc.load_gather(table_v, [sidx])
        plsc.store_scatter(table_v, [sidx], cur | _START_BIT)

    ctv0 = ct_v[pl.ds(0, _LANES)]
    sv0 = st_v[pl.ds(0, _LANES)]
    is0 = sv0 == 0
    ct_v[pl.ds(0, _LANES)] = jnp.where(is0, _START_BIT | _GID_MASK, ctv0)

    def issue(c, ei_v, b_v, sc_v, sem):
        chunk_id = wid + c * _NUM_TILES

        @pl.when(chunk_id < _N_CHUNKS)
        def _():
            base = pl.multiple_of(chunk_id * _CHUNK, _CHUNK)
            pltpu.async_copy(scores.at[pl.ds(base, _CHUNK)], sc_v, sem)
            pltpu.async_copy(ei.at[:, pl.ds(base, _CHUNK)], ei_v, sem)
            pltpu.async_copy(batch.at[pl.ds(base, _CHUNK)], b_v, sem)

    def process(c, ei_v, b_v, sc_v, sem, n_ei, n_b, n_sc, n_sem, o_v, osem):
        chunk_id = wid + c * _NUM_TILES

        @pl.when(chunk_id < _N_CHUNKS)
        def _():
            base = pl.multiple_of(chunk_id * _CHUNK, _CHUNK)
            issue(c + 1, n_ei, n_b, n_sc, n_sem)
            pltpu.make_async_copy(scores.at[pl.ds(0, _CHUNK)], sc_v, sem).wait()
            pltpu.make_async_copy(ei.at[:, pl.ds(0, _CHUNK)], ei_v, sem).wait()
            pltpu.make_async_copy(batch.at[pl.ds(0, _CHUNK)], b_v, sem).wait()

            @pl.when(chunk_id >= 2 * _NUM_TILES)
            def _():
                pltpu.make_async_copy(
                    o_v, out.at[pl.ds(0, _CHUNK)], osem).wait()

            def emit(off, ctv, ptv):
                h = ei_v[0, pl.ds(off, _LANES)]
                t = ei_v[1, pl.ds(off, _LANES)]
                sc = sc_v[pl.ds(off, _LANES)]
                hv = plsc.load_gather(table_v, [h])
                tv = plsc.load_gather(table_v, [t])
                cm = ctv & _GID_MASK
                hm = hv & _GID_MASK
                tm = tv & _GID_MASK
                hc = hm == cm
                tc = tm == cm
                hp = hm == ptv
                tp = tm == ptv
                valid = (hc | tc) & jnp.logical_not((hc & tp) | (tc & hp))
                is_start = ((hv | tv) & _START_BIT) != 0
                mask = jnp.where(ctv >= _START_BIT, is_start, valid)
                o_v[pl.ds(off, _LANES)] = jnp.where(mask, sc,
                                                    jnp.float32(-1e9))

            b0 = b_v[pl.ds(0, _LANES)]
            bl = b_v[pl.ds(_CHUNK - _LANES, _LANES)]
            uniform = jnp.all(b0 == bl)

            @pl.when(uniform)
            def _():
                ctv = plsc.load_gather(ct_v, [b0])
                ptv = plsc.load_gather(pt_v, [b0])

                @plsc.parallel_loop(0, _CHUNK, _LANES, unroll=10)
                def vec_body(off):
                    emit(off, ctv, ptv)

            @pl.when(jnp.logical_not(uniform))
            def _():
                @plsc.parallel_loop(0, _CHUNK, _LANES, unroll=5)
                def vec_body(off):
                    b = b_v[pl.ds(off, _LANES)]
                    emit(off, plsc.load_gather(ct_v, [b]),
                         plsc.load_gather(pt_v, [b]))

            pltpu.async_copy(o_v, out.at[pl.ds(base, _CHUNK)], osem)

    issue(0, ei_v0, b_v0, sc_v0, sem0)

    def superstep(g, carry):
        c = 2 * g
        process(c, ei_v0, b_v0, sc_v0, sem0,
                ei_v1, b_v1, sc_v1, sem1, o_v0, osem0)
        process(c + 1, ei_v1, b_v1, sc_v1, sem1,
                ei_v0, b_v0, sc_v0, sem0, o_v1, osem1)
        return carry

    lax.fori_loop(0, _ROUNDS // 2, superstep, 0)

    pltpu.make_async_copy(o_v0, out.at[pl.ds(0, _CHUNK)], osem0).wait()
    pltpu.make_async_copy(o_v1, out.at[pl.ds(0, _CHUNK)], osem1).wait()


def kernel(edge_scores, edge_index, edge_batch, node_global_ids,
           start_node_locals, current_tail, prev_tail, step_counts):
    mesh = plsc.VectorSubcoreMesh(core_axis_name="c", subcore_axis_name="s")
    run = pl.kernel(
        _body,
        mesh=mesh,
        compiler_params=pltpu.CompilerParams(needs_layout_passes=False),
        out_type=jax.ShapeDtypeStruct((_N_EDGES,), jnp.float32),
        scratch_types=[
            pltpu.VMEM((_N_NODES,), jnp.int32),
            pltpu.VMEM((_N_STARTS,), jnp.int32),
            pltpu.VMEM((_N_GRAPHS,), jnp.int32),
            pltpu.VMEM((_N_GRAPHS,), jnp.int32),
            pltpu.VMEM((_N_GRAPHS,), jnp.int32),
            pltpu.VMEM((2, _CHUNK), jnp.int32),
            pltpu.VMEM((_CHUNK,), jnp.int32),
            pltpu.VMEM((_CHUNK,), jnp.float32),
            pltpu.VMEM((2, _CHUNK), jnp.int32),
            pltpu.VMEM((_CHUNK,), jnp.int32),
            pltpu.VMEM((_CHUNK,), jnp.float32),
            pltpu.VMEM((_CHUNK,), jnp.float32),
            pltpu.VMEM((_CHUNK,), jnp.float32),
            pltpu.SemaphoreType.DMA,
            pltpu.SemaphoreType.DMA,
            pltpu.SemaphoreType.DMA,
            pltpu.SemaphoreType.DMA,
        ],
    )
    return run(edge_scores, edge_index, edge_batch, node_global_ids,
               start_node_locals, current_tail, prev_tail, step_counts)


# speedup vs baseline: 1.7034x; 1.7034x over previous
import jax
import jax.numpy as jnp
from jax import lax
from jax.experimental import pallas as pl
from jax.experimental.pallas import tpu as pltpu
from jax.experimental.pallas import tpu_sc as plsc

_N_NODES = 50000
_N_EDGES = 1600000
_N_GRAPHS = 16
_N_STARTS = 64
_LANES = 16
_NUM_TILES = 32
_CHUNK = 6400
_N_CHUNKS = _N_EDGES // _CHUNK
_ROUNDS = -(-_N_CHUNKS // _NUM_TILES)
_START_BIT = 1 << 30
_GID_MASK = _START_BIT - 1


def _body(scores, ei, batch, gid, starts, ct, pt, stc, out,
          table_v, starts_v, ct_v, pt_v, st_v,
          ei_v0, b_v0, sc_v0, ei_v1, b_v1, sc_v1, o_v0, o_v1,
          sem0, sem1, osem0, osem1):
    wid = lax.axis_index("s") * 2 + lax.axis_index("c")

    pltpu.sync_copy(gid, table_v)
    pltpu.sync_copy(starts, starts_v)
    pltpu.sync_copy(ct, ct_v)
    pltpu.sync_copy(pt, pt_v)
    pltpu.sync_copy(stc, st_v)

    for j in range(_N_STARTS // _LANES):
        sidx = starts_v[pl.ds(j * _LANES, _LANES)]
        cur = plsc.load_gather(table_v, [sidx])
        plsc.store_scatter(table_v, [sidx], cur | _START_BIT)

    ctv0 = ct_v[pl.ds(0, _LANES)]
    sv0 = st_v[pl.ds(0, _LANES)]
    is0 = sv0 == 0
    ct_v[pl.ds(0, _LANES)] = jnp.where(is0, _START_BIT | _GID_MASK, ctv0)

    def issue(c, ei_v, b_v, sc_v, sem):
        chunk_id = wid + c * _NUM_TILES

        @pl.when(chunk_id < _N_CHUNKS)
        def _():
            base = pl.multiple_of(chunk_id * _CHUNK, _CHUNK)
            pltpu.async_copy(scores.at[pl.ds(base, _CHUNK)], sc_v, sem)
            pltpu.async_copy(ei.at[:, pl.ds(base, _CHUNK)], ei_v, sem)
            pltpu.async_copy(batch.at[pl.ds(base, _CHUNK)], b_v, sem)

    def process(c, ei_v, b_v, sc_v, sem, n_ei, n_b, n_sc, n_sem, o_v, osem):
        chunk_id = wid + c * _NUM_TILES

        @pl.when(chunk_id < _N_CHUNKS)
        def _():
            base = pl.multiple_of(chunk_id * _CHUNK, _CHUNK)
            issue(c + 1, n_ei, n_b, n_sc, n_sem)
            pltpu.make_async_copy(scores.at[pl.ds(0, _CHUNK)], sc_v, sem).wait()
            pltpu.make_async_copy(ei.at[:, pl.ds(0, _CHUNK)], ei_v, sem).wait()
            pltpu.make_async_copy(batch.at[pl.ds(0, _CHUNK)], b_v, sem).wait()

            @pl.when(chunk_id >= 2 * _NUM_TILES)
            def _():
                pltpu.make_async_copy(
                    o_v, out.at[pl.ds(0, _CHUNK)], osem).wait()

            def emit(off, ctv, ptv):
                h = ei_v[0, pl.ds(off, _LANES)]
                t = ei_v[1, pl.ds(off, _LANES)]
                sc = sc_v[pl.ds(off, _LANES)]
                hv = plsc.load_gather(table_v, [h])
                tv = plsc.load_gather(table_v, [t])
                cm = ctv & _GID_MASK
                hm = hv & _GID_MASK
                tm = tv & _GID_MASK
                hc = hm == cm
                tc = tm == cm
                hp = hm == ptv
                tp = tm == ptv
                valid = (hc | tc) & jnp.logical_not((hc & tp) | (tc & hp))
                is_start = ((hv | tv) & _START_BIT) != 0
                mask = jnp.where(ctv >= _START_BIT, is_start, valid)
                o_v[pl.ds(off, _LANES)] = jnp.where(mask, sc,
                                                    jnp.float32(-1e9))

            b0 = b_v[pl.ds(0, _LANES)]
            bl = b_v[pl.ds(_CHUNK - _LANES, _LANES)]
            uniform = jnp.all(b0 == bl)

            @pl.when(uniform)
            def _():
                ctv = plsc.load_gather(ct_v, [b0])
                ptv = plsc.load_gather(pt_v, [b0])

                @plsc.parallel_loop(0, _CHUNK, _LANES, unroll=4)
                def vec_body(off):
                    emit(off, ctv, ptv)

            @pl.when(jnp.logical_not(uniform))
            def _():
                @plsc.parallel_loop(0, _CHUNK, _LANES, unroll=5)
                def vec_body(off):
                    b = b_v[pl.ds(off, _LANES)]
                    emit(off, plsc.load_gather(ct_v, [b]),
                         plsc.load_gather(pt_v, [b]))

            pltpu.async_copy(o_v, out.at[pl.ds(base, _CHUNK)], osem)

    issue(0, ei_v0, b_v0, sc_v0, sem0)

    def superstep(g, carry):
        c = 2 * g
        process(c, ei_v0, b_v0, sc_v0, sem0,
                ei_v1, b_v1, sc_v1, sem1, o_v0, osem0)
        process(c + 1, ei_v1, b_v1, sc_v1, sem1,
                ei_v0, b_v0, sc_v0, sem0, o_v1, osem1)
        return carry

    lax.fori_loop(0, _ROUNDS // 2, superstep, 0)

    pltpu.make_async_copy(o_v0, out.at[pl.ds(0, _CHUNK)], osem0).wait()
    pltpu.make_async_copy(o_v1, out.at[pl.ds(0, _CHUNK)], osem1).wait()


def kernel(edge_scores, edge_index, edge_batch, node_global_ids,
           start_node_locals, current_tail, prev_tail, step_counts):
    mesh = plsc.VectorSubcoreMesh(core_axis_name="c", subcore_axis_name="s")
    run = pl.kernel(
        _body,
        mesh=mesh,
        compiler_params=pltpu.CompilerParams(needs_layout_passes=False),
        out_type=jax.ShapeDtypeStruct((_N_EDGES,), jnp.float32),
        scratch_types=[
            pltpu.VMEM((_N_NODES,), jnp.int32),
            pltpu.VMEM((_N_STARTS,), jnp.int32),
            pltpu.VMEM((_N_GRAPHS,), jnp.int32),
            pltpu.VMEM((_N_GRAPHS,), jnp.int32),
            pltpu.VMEM((_N_GRAPHS,), jnp.int32),
            pltpu.VMEM((2, _CHUNK), jnp.int32),
            pltpu.VMEM((_CHUNK,), jnp.int32),
            pltpu.VMEM((_CHUNK,), jnp.float32),
            pltpu.VMEM((2, _CHUNK), jnp.int32),
            pltpu.VMEM((_CHUNK,), jnp.int32),
            pltpu.VMEM((_CHUNK,), jnp.float32),
            pltpu.VMEM((_CHUNK,), jnp.float32),
            pltpu.VMEM((_CHUNK,), jnp.float32),
            pltpu.SemaphoreType.DMA,
            pltpu.SemaphoreType.DMA,
            pltpu.SemaphoreType.DMA,
            pltpu.SemaphoreType.DMA,
        ],
    )
    return run(edge_scores, edge_index, edge_batch, node_global_ids,
               start_node_locals, current_tail, prev_tail, step_counts)


# speedup vs baseline: 1.8710x; 1.0984x over previous
import jax
import jax.numpy as jnp
from jax import lax
from jax.experimental import pallas as pl
from jax.experimental.pallas import tpu as pltpu
from jax.experimental.pallas import tpu_sc as plsc

_N_NODES = 50000
_N_EDGES = 1600000
_N_GRAPHS = 16
_N_STARTS = 64
_LANES = 16
_NUM_TILES = 32
_CHUNK = 6400
_N_CHUNKS = _N_EDGES // _CHUNK
_ROUNDS = -(-_N_CHUNKS // _NUM_TILES)
_START_BIT = 1 << 30
_GID_MASK = _START_BIT - 1


def _body(scores, ei, batch, gid, starts, ct, pt, stc, out,
          table_v, starts_v, ct_v, pt_v, st_v,
          ei_v0, b_v0, sc_v0, ei_v1, b_v1, sc_v1, o_v0, o_v1,
          sem0, sem1, osem0, osem1):
    wid = lax.axis_index("s") * 2 + lax.axis_index("c")

    def issue(c, ei_v, b_v, sc_v, sem):
        chunk_id = wid + c * _NUM_TILES

        @pl.when(chunk_id < _N_CHUNKS)
        def _():
            base = pl.multiple_of(chunk_id * _CHUNK, _CHUNK)
            pltpu.async_copy(scores.at[pl.ds(base, _CHUNK)], sc_v, sem)
            pltpu.async_copy(ei.at[:, pl.ds(base, _CHUNK)], ei_v, sem)
            pltpu.async_copy(batch.at[pl.ds(base, _CHUNK)], b_v, sem)

    p0 = pltpu.async_copy(gid, table_v, osem0)
    p1 = pltpu.async_copy(starts, starts_v, osem0)
    p2 = pltpu.async_copy(ct, ct_v, osem0)
    p3 = pltpu.async_copy(pt, pt_v, osem0)
    p4 = pltpu.async_copy(stc, st_v, osem0)
    issue(0, ei_v0, b_v0, sc_v0, sem0)
    p0.wait()
    p1.wait()
    p2.wait()
    p3.wait()
    p4.wait()

    for j in range(_N_STARTS // _LANES):
        sidx = starts_v[pl.ds(j * _LANES, _LANES)]
        cur = plsc.load_gather(table_v, [sidx])
        plsc.store_scatter(table_v, [sidx], cur | _START_BIT)

    ctv0 = ct_v[pl.ds(0, _LANES)]
    sv0 = st_v[pl.ds(0, _LANES)]
    is0 = sv0 == 0
    ct_v[pl.ds(0, _LANES)] = jnp.where(is0, _START_BIT | _GID_MASK, ctv0)

    def process(c, ei_v, b_v, sc_v, sem, n_ei, n_b, n_sc, n_sem, o_v, osem):
        chunk_id = wid + c * _NUM_TILES

        @pl.when(chunk_id < _N_CHUNKS)
        def _():
            base = pl.multiple_of(chunk_id * _CHUNK, _CHUNK)
            issue(c + 1, n_ei, n_b, n_sc, n_sem)
            pltpu.make_async_copy(scores.at[pl.ds(0, _CHUNK)], sc_v, sem).wait()
            pltpu.make_async_copy(ei.at[:, pl.ds(0, _CHUNK)], ei_v, sem).wait()
            pltpu.make_async_copy(batch.at[pl.ds(0, _CHUNK)], b_v, sem).wait()

            @pl.when(chunk_id >= 2 * _NUM_TILES)
            def _():
                pltpu.make_async_copy(
                    o_v, out.at[pl.ds(0, _CHUNK)], osem).wait()

            def emit(off, ctv, ptv):
                h = ei_v[0, pl.ds(off, _LANES)]
                t = ei_v[1, pl.ds(off, _LANES)]
                sc = sc_v[pl.ds(off, _LANES)]
                hv = plsc.load_gather(table_v, [h])
                tv = plsc.load_gather(table_v, [t])
                cm = ctv & _GID_MASK
                hm = hv & _GID_MASK
                tm = tv & _GID_MASK
                hc = hm == cm
                tc = tm == cm
                hp = hm == ptv
                tp = tm == ptv
                valid = (hc | tc) & jnp.logical_not((hc & tp) | (tc & hp))
                is_start = ((hv | tv) & _START_BIT) != 0
                mask = jnp.where(ctv >= _START_BIT, is_start, valid)
                o_v[pl.ds(off, _LANES)] = jnp.where(mask, sc,
                                                    jnp.float32(-1e9))

            b0 = b_v[pl.ds(0, _LANES)]
            bl = b_v[pl.ds(_CHUNK - _LANES, _LANES)]
            uniform = jnp.all(b0 == bl)

            @pl.when(uniform)
            def _():
                ctv = plsc.load_gather(ct_v, [b0])
                ptv = plsc.load_gather(pt_v, [b0])

                @plsc.parallel_loop(0, _CHUNK, _LANES, unroll=5)
                def vec_body(off):
                    emit(off, ctv, ptv)

            @pl.when(jnp.logical_not(uniform))
            def _():
                @plsc.parallel_loop(0, _CHUNK, _LANES, unroll=5)
                def vec_body(off):
                    b = b_v[pl.ds(off, _LANES)]
                    emit(off, plsc.load_gather(ct_v, [b]),
                         plsc.load_gather(pt_v, [b]))

            pltpu.async_copy(o_v, out.at[pl.ds(base, _CHUNK)], osem)


    def superstep(g, carry):
        c = 2 * g
        process(c, ei_v0, b_v0, sc_v0, sem0,
                ei_v1, b_v1, sc_v1, sem1, o_v0, osem0)
        process(c + 1, ei_v1, b_v1, sc_v1, sem1,
                ei_v0, b_v0, sc_v0, sem0, o_v1, osem1)
        return carry

    lax.fori_loop(0, _ROUNDS // 2, superstep, 0)

    pltpu.make_async_copy(o_v0, out.at[pl.ds(0, _CHUNK)], osem0).wait()
    pltpu.make_async_copy(o_v1, out.at[pl.ds(0, _CHUNK)], osem1).wait()


def kernel(edge_scores, edge_index, edge_batch, node_global_ids,
           start_node_locals, current_tail, prev_tail, step_counts):
    mesh = plsc.VectorSubcoreMesh(core_axis_name="c", subcore_axis_name="s")
    run = pl.kernel(
        _body,
        mesh=mesh,
        compiler_params=pltpu.CompilerParams(needs_layout_passes=False),
        out_type=jax.ShapeDtypeStruct((_N_EDGES,), jnp.float32),
        scratch_types=[
            pltpu.VMEM((_N_NODES,), jnp.int32),
            pltpu.VMEM((_N_STARTS,), jnp.int32),
            pltpu.VMEM((_N_GRAPHS,), jnp.int32),
            pltpu.VMEM((_N_GRAPHS,), jnp.int32),
            pltpu.VMEM((_N_GRAPHS,), jnp.int32),
            pltpu.VMEM((2, _CHUNK), jnp.int32),
            pltpu.VMEM((_CHUNK,), jnp.int32),
            pltpu.VMEM((_CHUNK,), jnp.float32),
            pltpu.VMEM((2, _CHUNK), jnp.int32),
            pltpu.VMEM((_CHUNK,), jnp.int32),
            pltpu.VMEM((_CHUNK,), jnp.float32),
            pltpu.VMEM((_CHUNK,), jnp.float32),
            pltpu.VMEM((_CHUNK,), jnp.float32),
            pltpu.SemaphoreType.DMA,
            pltpu.SemaphoreType.DMA,
            pltpu.SemaphoreType.DMA,
            pltpu.SemaphoreType.DMA,
        ],
    )
    return run(edge_scores, edge_index, edge_batch, node_global_ids,
               start_node_locals, current_tail, prev_tail, step_counts)


# speedup vs baseline: 1.8730x; 1.0011x over previous
import jax
import jax.numpy as jnp
from jax import lax
from jax.experimental import pallas as pl
from jax.experimental.pallas import tpu as pltpu
from jax.experimental.pallas import tpu_sc as plsc

_N_NODES = 50000
_N_EDGES = 1600000
_N_GRAPHS = 16
_N_STARTS = 64
_LANES = 16
_NUM_TILES = 32
_CHUNK = 6400
_N_CHUNKS = _N_EDGES // _CHUNK
_ROUNDS = -(-_N_CHUNKS // _NUM_TILES)
_START_BIT = 1 << 30
_GID_MASK = _START_BIT - 1


def _body(scores, ei, batch, gid, starts, ct, pt, stc, out,
          table_v, starts_v, ct_v, pt_v, st_v,
          ei_v0, b_v0, sc_v0, ei_v1, b_v1, sc_v1, o_v0, o_v1,
          sem0, sem1, osem0, osem1):
    wid = lax.axis_index("s") * 2 + lax.axis_index("c")

    def issue(c, ei_v, b_v, sc_v, sem):
        chunk_id = wid + c * _NUM_TILES

        @pl.when(chunk_id < _N_CHUNKS)
        def _():
            base = pl.multiple_of(chunk_id * _CHUNK, _CHUNK)
            pltpu.async_copy(scores.at[pl.ds(base, _CHUNK)], sc_v, sem)
            pltpu.async_copy(ei.at[:, pl.ds(base, _CHUNK)], ei_v, sem)
            pltpu.async_copy(batch.at[pl.ds(base, _CHUNK)], b_v, sem)

    p0 = pltpu.async_copy(gid, table_v, osem0)
    p1 = pltpu.async_copy(starts, starts_v, osem0)
    p2 = pltpu.async_copy(ct, ct_v, osem0)
    p3 = pltpu.async_copy(pt, pt_v, osem0)
    p4 = pltpu.async_copy(stc, st_v, osem0)
    issue(0, ei_v0, b_v0, sc_v0, sem0)
    p0.wait()
    p1.wait()
    p2.wait()
    p3.wait()
    p4.wait()

    for j in range(_N_STARTS // _LANES):
        sidx = starts_v[pl.ds(j * _LANES, _LANES)]
        cur = plsc.load_gather(table_v, [sidx])
        plsc.store_scatter(table_v, [sidx], cur | _START_BIT)

    ctv0 = ct_v[pl.ds(0, _LANES)]
    sv0 = st_v[pl.ds(0, _LANES)]
    is0 = sv0 == 0
    ct_v[pl.ds(0, _LANES)] = jnp.where(is0, _START_BIT | _GID_MASK, ctv0)

    def process(c, ei_v, b_v, sc_v, sem, n_ei, n_b, n_sc, n_sem, o_v, osem):
        chunk_id = wid + c * _NUM_TILES

        @pl.when(chunk_id < _N_CHUNKS)
        def _():
            base = pl.multiple_of(chunk_id * _CHUNK, _CHUNK)
            issue(c + 1, n_ei, n_b, n_sc, n_sem)
            pltpu.make_async_copy(scores.at[pl.ds(0, _CHUNK)], sc_v, sem).wait()
            pltpu.make_async_copy(ei.at[:, pl.ds(0, _CHUNK)], ei_v, sem).wait()
            pltpu.make_async_copy(batch.at[pl.ds(0, _CHUNK)], b_v, sem).wait()

            @pl.when(chunk_id >= 2 * _NUM_TILES)
            def _():
                pltpu.make_async_copy(
                    o_v, out.at[pl.ds(0, _CHUNK)], osem).wait()

            def emit(off, ctv, ptv):
                h = ei_v[0, pl.ds(off, _LANES)]
                t = ei_v[1, pl.ds(off, _LANES)]
                sc = sc_v[pl.ds(off, _LANES)]
                hv = plsc.load_gather(table_v, [h])
                tv = plsc.load_gather(table_v, [t])
                cm = ctv & _GID_MASK
                hm = hv & _GID_MASK
                tm = tv & _GID_MASK
                hc = hm == cm
                tc = tm == cm
                hp = hm == ptv
                tp = tm == ptv
                valid = (hc | tc) & jnp.logical_not((hc & tp) | (tc & hp))
                is_start = ((hv | tv) & _START_BIT) != 0
                mask = jnp.where(ctv >= _START_BIT, is_start, valid)
                o_v[pl.ds(off, _LANES)] = jnp.where(mask, sc,
                                                    jnp.float32(-1e9))

            b0 = b_v[pl.ds(0, _LANES)]
            bl = b_v[pl.ds(_CHUNK - _LANES, _LANES)]
            uniform = jnp.all(b0 == bl)

            @pl.when(uniform)
            def _():
                ctv = plsc.load_gather(ct_v, [b0])
                ptv = plsc.load_gather(pt_v, [b0])

                @plsc.parallel_loop(0, _CHUNK, _LANES, unroll=5)
                def vec_body(off):
                    emit(off, ctv, ptv)

            @pl.when(jnp.logical_not(uniform))
            def _():
                @plsc.parallel_loop(0, _CHUNK, _LANES, unroll=2)
                def vec_body(off):
                    b = b_v[pl.ds(off, _LANES)]
                    emit(off, plsc.load_gather(ct_v, [b]),
                         plsc.load_gather(pt_v, [b]))

            pltpu.async_copy(o_v, out.at[pl.ds(base, _CHUNK)], osem)


    def superstep(g, carry):
        c = 2 * g
        process(c, ei_v0, b_v0, sc_v0, sem0,
                ei_v1, b_v1, sc_v1, sem1, o_v0, osem0)
        process(c + 1, ei_v1, b_v1, sc_v1, sem1,
                ei_v0, b_v0, sc_v0, sem0, o_v1, osem1)
        return carry

    lax.fori_loop(0, _ROUNDS // 2, superstep, 0)

    pltpu.make_async_copy(o_v0, out.at[pl.ds(0, _CHUNK)], osem0).wait()
    pltpu.make_async_copy(o_v1, out.at[pl.ds(0, _CHUNK)], osem1).wait()


def kernel(edge_scores, edge_index, edge_batch, node_global_ids,
           start_node_locals, current_tail, prev_tail, step_counts):
    mesh = plsc.VectorSubcoreMesh(core_axis_name="c", subcore_axis_name="s")
    run = pl.kernel(
        _body,
        mesh=mesh,
        compiler_params=pltpu.CompilerParams(needs_layout_passes=False),
        out_type=jax.ShapeDtypeStruct((_N_EDGES,), jnp.float32),
        scratch_types=[
            pltpu.VMEM((_N_NODES,), jnp.int32),
            pltpu.VMEM((_N_STARTS,), jnp.int32),
            pltpu.VMEM((_N_GRAPHS,), jnp.int32),
            pltpu.VMEM((_N_GRAPHS,), jnp.int32),
            pltpu.VMEM((_N_GRAPHS,), jnp.int32),
            pltpu.VMEM((2, _CHUNK), jnp.int32),
            pltpu.VMEM((_CHUNK,), jnp.int32),
            pltpu.VMEM((_CHUNK,), jnp.float32),
            pltpu.VMEM((2, _CHUNK), jnp.int32),
            pltpu.VMEM((_CHUNK,), jnp.int32),
            pltpu.VMEM((_CHUNK,), jnp.float32),
            pltpu.VMEM((_CHUNK,), jnp.float32),
            pltpu.VMEM((_CHUNK,), jnp.float32),
            pltpu.SemaphoreType.DMA,
            pltpu.SemaphoreType.DMA,
            pltpu.SemaphoreType.DMA,
            pltpu.SemaphoreType.DMA,
        ],
    )
    return run(edge_scores, edge_index, edge_batch, node_global_ids,
               start_node_locals, current_tail, prev_tail, step_counts)
